# baseline (speedup 1.0000x reference)
import jax
import jax.numpy as jnp
from jax import lax
from jax.experimental import pallas as pl
from jax.experimental.pallas import tpu as pltpu

_GAMMA = 2.0
_SC = 0.0001
_ALPHA = 0.25

_N = 20000
_C = 80
_G = 128
_BN = 5000
_NB = _N // _BN

_DN_STD = (((1,), (0,)), ((), ()))
_DN_TT = (((0,), (0,)), ((), ()))


def _mask_loss_kernel(lab_ref, logits_ref, scores_ref, iib_ref, iou_ref,
                      npv_ref, out_ref, la, ab, conf, macc, negacc, coracc,
                      acc1, acc2, acc3):
    s = pl.program_id(0)
    i = pl.program_id(1)

    @pl.when((s == 0) & (i == 0))
    def _init():
        conf[...] = jnp.zeros_like(conf)
        macc[...] = jnp.zeros_like(macc)
        negacc[...] = jnp.zeros_like(negacc)
        coracc[...] = jnp.zeros_like(coracc)
        acc1[...] = jnp.zeros_like(acc1)
        acc2[...] = jnp.zeros_like(acc2)
        acc3[...] = jnp.zeros_like(acc3)

    @pl.when(s == 0)
    def _stage0():
        onesr = jnp.ones((1, _BN), jnp.float32)
        lb = logits_ref[...]
        bp = jnp.clip(jax.nn.sigmoid(lb), _SC, 1.0 - _SC)
        onem = 1.0 - bp
        l1m = jnp.log(onem)
        fneg = l1m * bp * bp
        negacc[...] += lax.dot_general(
            onesr, fneg, _DN_STD, preferred_element_type=jnp.float32)
        iib = iib_ref[...]
        iibf = iib.astype(jnp.float32)
        mask = iib > 0
        kk = lax.dot_general(iibf, jnp.ones((_G, 1), jnp.float32), _DN_STD,
                             preferred_element_type=jnp.float32)
        b = fneg[:, 0:1]
        logp0 = jnp.clip(lb[:, 0:1] + l1m[:, 0:1],
                         jnp.log(_SC), jnp.log(1.0 - _SC))
        a = logp0 * onem[:, 0:1] * onem[:, 0:1]
        selb = jnp.where(kk > 0.0, b, 0.0)
        coracc[...] += lax.dot_general(
            onesr, selb, _DN_STD, preferred_element_type=jnp.float32)
        ab2 = jnp.concatenate([a, b], axis=1)
        ab[pl.ds(i * _BN, _BN), :] = ab2
        acc3[...] += lax.dot_general(
            ab2, iibf, _DN_TT, preferred_element_type=jnp.float32)
        oh = (lab_ref[...] ==
              jax.lax.broadcasted_iota(jnp.int32, (_C, _G), 0)
              ).astype(jnp.float32)
        align = lax.dot_general(scores_ref[...], oh, _DN_STD,
                                preferred_element_type=jnp.float32) * iibf
        conf[...] = jnp.maximum(conf[...],
                                jnp.max(align, axis=0, keepdims=True))
        la[pl.ds(i * _BN, _BN), :] = jnp.where(
            mask, jnp.log(jnp.maximum(align, 1e-38)), -1e9)

    @pl.when(s == 1)
    def _stage1():
        oh = (lab_ref[...] ==
              jax.lax.broadcasted_iota(jnp.int32, (_C, _G), 0)
              ).astype(jnp.float32)
        giou = lax.dot_general(iou_ref[...], oh, _DN_STD,
                               preferred_element_type=jnp.float32)
        lab = la[pl.ds(i * _BN, _BN), :]
        p = jnp.exp(conf[...] * lab)
        mpw = p * giou
        mpw2 = mpw * mpw
        macc[...] = jnp.maximum(macc[...],
                                jnp.max(mpw, axis=0, keepdims=True))
        abb = ab[pl.ds(i * _BN, _BN), :]
        acc1[...] += lax.dot_general(
            abb, mpw2, _DN_TT, preferred_element_type=jnp.float32)
        acc2[...] += lax.dot_general(
            abb, mpw, _DN_TT, preferred_element_type=jnp.float32)

    @pl.when((s == 1) & (i == _NB - 1))
    def _final():
        rinv = 1.0 / (macc[...] + _SC)
        rinv2 = rinv * rinv
        a2q = acc1[0:1, :] + 2.0 * _SC * acc2[0:1, :] + _SC * _SC * acc3[0:1, :]
        k1q = acc2[1:2, :] + _SC * acc3[1:2, :]
        k2q = acc1[1:2, :] + 2.0 * _SC * acc2[1:2, :] + _SC * _SC * acc3[1:2, :]
        posv = a2q * rinv2
        bnegv = acc3[1:2, :] - 2.0 * k1q * rinv + k2q * rinv2
        sneg = -jnp.sum(negacc[...], axis=1, keepdims=True)
        spos = jnp.sum(posv, axis=1, keepdims=True)
        sbneg = jnp.sum(bnegv, axis=1, keepdims=True)
        total = ((1.0 - _ALPHA) * (sneg + coracc[...])
                 - _ALPHA * (spos + sbneg))
        out_ref[...] = total / npv_ref[...]


def kernel(logits_pred, scores, iou_map, is_in_boxes, labels, num_pos_avg):
    out = pl.pallas_call(
        _mask_loss_kernel,
        grid=(2, _NB),
        in_specs=[
            pl.BlockSpec((1, _G), lambda s, i: (0, 0)),
            pl.BlockSpec((_BN, _C),
                         lambda s, i: (jnp.where(s == 0, i, _NB - 1), 0)),
            pl.BlockSpec((_BN, _C),
                         lambda s, i: (jnp.where(s == 0, i, _NB - 1), 0)),
            pl.BlockSpec((_BN, _G),
                         lambda s, i: (jnp.where(s == 0, i, _NB - 1), 0)),
            pl.BlockSpec((_BN, _C), lambda s, i: (jnp.where(s == 1, i, 0), 0)),
            pl.BlockSpec((1, 1), lambda s, i: (0, 0)),
        ],
        out_specs=pl.BlockSpec((1, 1), lambda s, i: (0, 0)),
        out_shape=jax.ShapeDtypeStruct((1, 1), jnp.float32),
        scratch_shapes=[
            pltpu.VMEM((_N, _G), jnp.float32),
            pltpu.VMEM((_N, 2), jnp.float32),
            pltpu.VMEM((1, _G), jnp.float32),
            pltpu.VMEM((1, _G), jnp.float32),
            pltpu.VMEM((1, _C), jnp.float32),
            pltpu.VMEM((1, 1), jnp.float32),
            pltpu.VMEM((2, _G), jnp.float32),
            pltpu.VMEM((2, _G), jnp.float32),
            pltpu.VMEM((2, _G), jnp.float32),
        ],
        compiler_params=pltpu.CompilerParams(
            dimension_semantics=("arbitrary", "arbitrary")),
    )(labels[None, :], logits_pred, scores, is_in_boxes, iou_map,
      jnp.asarray(num_pos_avg, jnp.float32).reshape(1, 1))
    return out[0, 0]


# speedup vs baseline: 1.1062x; 1.1062x over previous
import jax
import jax.numpy as jnp
from jax import lax
from jax.experimental import pallas as pl
from jax.experimental.pallas import tpu as pltpu

_GAMMA = 2.0
_SC = 0.0001
_ALPHA = 0.25

_N = 20000
_C = 80
_G = 128
_BN = 4000
_NB = _N // _BN

_DN_STD = (((1,), (0,)), ((), ()))
_DN_TT = (((0,), (0,)), ((), ()))


def _mask_loss_kernel(lab_ref, logits_ref, scores_ref, iib_ref, iou_ref,
                      npv_ref, out_ref, la, ab, conf, macc, negacc, coracc,
                      acc1, acc2, acc3):
    s = pl.program_id(0)
    i = pl.program_id(1)

    @pl.when((s == 0) & (i == 0))
    def _init():
        conf[...] = jnp.zeros_like(conf)
        macc[...] = jnp.zeros_like(macc)
        negacc[...] = jnp.zeros_like(negacc)
        coracc[...] = jnp.zeros_like(coracc)
        acc1[...] = jnp.zeros_like(acc1)
        acc2[...] = jnp.zeros_like(acc2)
        acc3[...] = jnp.zeros_like(acc3)

    @pl.when(s == 0)
    def _stage0():
        onesr = jnp.ones((1, _BN), jnp.float32)
        lb = logits_ref[...]
        bp = jnp.clip(jax.nn.sigmoid(lb), _SC, 1.0 - _SC)
        onem = 1.0 - bp
        l1m = jnp.log(onem)
        fneg = l1m * bp * bp
        negacc[...] += lax.dot_general(
            onesr, fneg, _DN_STD, preferred_element_type=jnp.float32)
        iib = iib_ref[...]
        iibf = iib.astype(jnp.float32)
        mask = iib > 0
        kk = lax.dot_general(iibf, jnp.ones((_G, 1), jnp.float32), _DN_STD,
                             preferred_element_type=jnp.float32)
        b = fneg[:, 0:1]
        logp0 = jnp.clip(lb[:, 0:1] + l1m[:, 0:1],
                         jnp.log(_SC), jnp.log(1.0 - _SC))
        a = logp0 * onem[:, 0:1] * onem[:, 0:1]
        selb = jnp.where(kk > 0.0, b, 0.0)
        coracc[...] += lax.dot_general(
            onesr, selb, _DN_STD, preferred_element_type=jnp.float32)
        ab2 = jnp.concatenate([a, b], axis=1)
        ab[pl.ds(i * _BN, _BN), :] = ab2
        acc3[...] += lax.dot_general(
            ab2, iibf, _DN_TT, preferred_element_type=jnp.float32)
        oh = (lab_ref[...] ==
              jax.lax.broadcasted_iota(jnp.int32, (_C, _G), 0)
              ).astype(jnp.float32)
        align = lax.dot_general(scores_ref[...], oh, _DN_STD,
                                preferred_element_type=jnp.float32) * iibf
        conf[...] = jnp.maximum(conf[...],
                                jnp.max(align, axis=0, keepdims=True))
        la[pl.ds(i * _BN, _BN), :] = jnp.where(
            mask, jnp.log(jnp.maximum(align, 1e-38)), -1e9)

    @pl.when(s == 1)
    def _stage1():
        oh = (lab_ref[...] ==
              jax.lax.broadcasted_iota(jnp.int32, (_C, _G), 0)
              ).astype(jnp.float32)
        giou = lax.dot_general(iou_ref[...], oh, _DN_STD,
                               preferred_element_type=jnp.float32)
        lab = la[pl.ds(i * _BN, _BN), :]
        p = jnp.exp(conf[...] * lab)
        mpw = p * giou
        mpw2 = mpw * mpw
        macc[...] = jnp.maximum(macc[...],
                                jnp.max(mpw, axis=0, keepdims=True))
        abb = ab[pl.ds(i * _BN, _BN), :]
        acc1[...] += lax.dot_general(
            abb, mpw2, _DN_TT, preferred_element_type=jnp.float32)
        acc2[...] += lax.dot_general(
            abb, mpw, _DN_TT, preferred_element_type=jnp.float32)

    @pl.when((s == 1) & (i == _NB - 1))
    def _final():
        rinv = 1.0 / (macc[...] + _SC)
        rinv2 = rinv * rinv
        a2q = acc1[0:1, :] + 2.0 * _SC * acc2[0:1, :] + _SC * _SC * acc3[0:1, :]
        k1q = acc2[1:2, :] + _SC * acc3[1:2, :]
        k2q = acc1[1:2, :] + 2.0 * _SC * acc2[1:2, :] + _SC * _SC * acc3[1:2, :]
        posv = a2q * rinv2
        bnegv = acc3[1:2, :] - 2.0 * k1q * rinv + k2q * rinv2
        sneg = -jnp.sum(negacc[...], axis=1, keepdims=True)
        spos = jnp.sum(posv, axis=1, keepdims=True)
        sbneg = jnp.sum(bnegv, axis=1, keepdims=True)
        total = ((1.0 - _ALPHA) * (sneg + coracc[...])
                 - _ALPHA * (spos + sbneg))
        out_ref[...] = total / npv_ref[...]


def kernel(logits_pred, scores, iou_map, is_in_boxes, labels, num_pos_avg):
    out = pl.pallas_call(
        _mask_loss_kernel,
        grid=(2, _NB),
        in_specs=[
            pl.BlockSpec((1, _G), lambda s, i: (0, 0)),
            pl.BlockSpec((_BN, _C),
                         lambda s, i: (jnp.where(s == 0, i, _NB - 1), 0)),
            pl.BlockSpec((_BN, _C),
                         lambda s, i: (jnp.where(s == 0, i, _NB - 1), 0)),
            pl.BlockSpec((_BN, _G),
                         lambda s, i: (jnp.where(s == 0, i, _NB - 1), 0)),
            pl.BlockSpec((_BN, _C), lambda s, i: (jnp.where(s == 1, i, 0), 0)),
            pl.BlockSpec((1, 1), lambda s, i: (0, 0)),
        ],
        out_specs=pl.BlockSpec((1, 1), lambda s, i: (0, 0)),
        out_shape=jax.ShapeDtypeStruct((1, 1), jnp.float32),
        scratch_shapes=[
            pltpu.VMEM((_N, _G), jnp.float32),
            pltpu.VMEM((_N, 2), jnp.float32),
            pltpu.VMEM((1, _G), jnp.float32),
            pltpu.VMEM((1, _G), jnp.float32),
            pltpu.VMEM((1, _C), jnp.float32),
            pltpu.VMEM((1, 1), jnp.float32),
            pltpu.VMEM((2, _G), jnp.float32),
            pltpu.VMEM((2, _G), jnp.float32),
            pltpu.VMEM((2, _G), jnp.float32),
        ],
        compiler_params=pltpu.CompilerParams(
            dimension_semantics=("arbitrary", "arbitrary")),
    )(labels[None, :], logits_pred, scores, is_in_boxes, iou_map,
      jnp.asarray(num_pos_avg, jnp.float32).reshape(1, 1))
    return out[0, 0]
